# baseline (speedup 1.0000x reference)
import functools

import jax
import jax.numpy as jnp
from jax import lax
from jax.experimental import pallas as pl
from jax.experimental.pallas import tpu as pltpu
from jax.experimental.pallas import tpu_sc as plsc

_E, _D, _FF, _K = 8, 1024, 2048, 2
_N = 2048
_NK = _N * _K
_TILE = 512
_T = _NK // _TILE
_W = _T + _E - 1
_FFT = 512
_F = _FF // _FFT

_INFO = plsc.get_sparse_core_info()
_NW = _INFO.num_cores * _INFO.num_subcores
_CHUNK = _N // _NW


def _cumsum(v, axis):
    n = v.shape[axis]
    sh = 1
    while sh < n:
        z = jnp.zeros_like(jax.lax.slice_in_dim(v, 0, sh, axis=axis))
        v = v + jnp.concatenate(
            [z, jax.lax.slice_in_dim(v, 0, n - sh, axis=axis)], axis=axis)
        sh *= 2
    return v


def _plan_kernel(x_ref, gw_ref, gb_ref, logits_ref, x3_ref,
                 g0_ref, g1_ref, p0_ref, p1_ref, off_ref, cnt_ref):
    x = x_ref[...]
    x3_ref[...] = x.reshape(_N, 8, 128)
    logits = jax.lax.dot_general(
        x, gw_ref[...], (((1,), (1,)), ((), ())),
        preferred_element_type=jnp.float32) + gb_ref[...]
    logits_ref[...] = logits
    lt = logits.T
    sub = jax.lax.broadcasted_iota(jnp.int32, lt.shape, 0)
    l0 = jnp.max(lt, axis=0, keepdims=True)
    i0 = jnp.min(jnp.where(lt == l0, sub, _E), axis=0, keepdims=True)
    masked = jnp.where(sub == i0, -jnp.inf, lt)
    l1 = jnp.max(masked, axis=0, keepdims=True)
    i1 = jnp.min(jnp.where(masked == l1, sub, _E), axis=0, keepdims=True)
    g0 = jax.nn.sigmoid(l0 - l1)
    lane = jax.lax.broadcasted_iota(jnp.int32, logits.shape, 1)
    l0c = jnp.max(logits, axis=1, keepdims=True)
    i0c = jnp.min(jnp.where(logits == l0c, lane, _E), axis=1, keepdims=True)
    mkc = jnp.where(lane == i0c, -jnp.inf, logits)
    l1c = jnp.max(mkc, axis=1, keepdims=True)
    g0c = jax.nn.sigmoid(l0c - l1c)
    g0_ref[...] = jnp.broadcast_to(g0c, (_N, 128))
    g1_ref[...] = jnp.broadcast_to(1.0 - g0c, (_N, 128))
    m0 = (sub == i0).astype(jnp.int32)
    m1 = (sub == i1).astype(jnp.int32)
    c0i = _cumsum(m0, 1)
    c1i = _cumsum(m1, 1)
    cnt0 = c0i[:, _N - 1:_N]
    cnt1 = c1i[:, _N - 1:_N]
    counts = cnt0 + cnt1
    off = _cumsum(counts, 0) - counts
    off_ref[...] = off
    cnt_ref[...] = counts
    p0 = jnp.sum(m0 * (off + c0i - m0), axis=0, keepdims=True)
    p1 = jnp.sum(m1 * (off + cnt0 + c1i - m1), axis=0, keepdims=True)
    p0_ref[...] = p0.reshape(_N)
    p1_ref[...] = p1.reshape(_N)


def _worklist_kernel(off_ref, cnt_ref, wt_ref, we_ref, wv_ref):
    def wbody(p, idx):
        t = p // _E
        e = p % _E
        s = off_ref[e, 0]
        c = cnt_ref[e, 0]
        active = jnp.logical_and(
            jnp.logical_and(s < (t + 1) * _TILE, s + c > t * _TILE), c > 0)

        @pl.when(active)
        def _():
            wt_ref[idx] = t
            we_ref[idx] = e
            wv_ref[idx] = 1

        return idx + active.astype(jnp.int32)

    nitems = jax.lax.fori_loop(0, _T * _E, wbody, 0)

    def pbody(i, carry):
        @pl.when(i >= nitems)
        def _():
            wt_ref[i] = wt_ref[nitems - 1]
            we_ref[i] = we_ref[nitems - 1]
            wv_ref[i] = 0
        return carry

    jax.lax.fori_loop(0, _W, pbody, 0)


_SC_MESH = plsc.VectorSubcoreMesh(core_axis_name="c", subcore_axis_name="s")


@functools.partial(
    pl.kernel, mesh=_SC_MESH,
    out_type=jax.ShapeDtypeStruct((_NK, 8, 128), jnp.float32),
    scratch_types=[
        pltpu.VMEM((_CHUNK,), jnp.int32),
        pltpu.VMEM((_CHUNK,), jnp.int32),
        pltpu.VMEM((_CHUNK, 8, 128), jnp.float32),
        pltpu.SemaphoreType.DMA,
    ],
)
def _sc_dispatch(x3_hbm, p0_hbm, p1_hbm, xs_hbm, i0_v, i1_v, rows_v, sem):
    wid = lax.axis_index("s") * _INFO.num_cores + lax.axis_index("c")
    base = wid * _CHUNK
    pltpu.sync_copy(p0_hbm.at[pl.ds(base, _CHUNK)], i0_v)
    pltpu.sync_copy(p1_hbm.at[pl.ds(base, _CHUNK)], i1_v)
    pltpu.async_copy(x3_hbm.at[pl.ds(base, _CHUNK)], rows_v, sem).wait()
    pltpu.sync_copy(rows_v, xs_hbm.at[i0_v])
    pltpu.sync_copy(rows_v, xs_hbm.at[i1_v])


_CC = 16


@functools.partial(
    pl.kernel, mesh=_SC_MESH,
    out_type=jax.ShapeDtypeStruct((_N, 8, 128), jnp.float32),
    scratch_types=[
        pltpu.VMEM((_CC,), jnp.int32),
        pltpu.VMEM((_CC,), jnp.int32),
        pltpu.VMEM((_CC, 128), jnp.float32),
        pltpu.VMEM((_CC, 128), jnp.float32),
        pltpu.VMEM((_CC, 8, 128), jnp.float32),
        pltpu.VMEM((_CC, 8, 128), jnp.float32),
        pltpu.VMEM((_CC, 8, 128), jnp.float32),
        pltpu.SemaphoreType.DMA,
        pltpu.SemaphoreType.DMA,
    ],
)
def _sc_combine(os_hbm, p0_hbm, p1_hbm, g0_hbm, g1_hbm, out_hbm,
                i0_v, i1_v, g0_v, g1_v, r0_v, r1_v, o_v, sem0, sem1):
    wid = lax.axis_index("s") * _INFO.num_cores + lax.axis_index("c")
    for cc in range(_CHUNK // _CC):
        base = wid * _CHUNK + cc * _CC
        pltpu.sync_copy(p0_hbm.at[pl.ds(base, _CC)], i0_v)
        pltpu.sync_copy(p1_hbm.at[pl.ds(base, _CC)], i1_v)
        pltpu.sync_copy(g0_hbm.at[pl.ds(base, _CC)], g0_v)
        pltpu.sync_copy(g1_hbm.at[pl.ds(base, _CC)], g1_v)
        c0 = pltpu.async_copy(os_hbm.at[i0_v], r0_v, sem0)
        c1 = pltpu.async_copy(os_hbm.at[i1_v], r1_v, sem1)
        c0.wait()
        c1.wait()

        def tok(tt, carry):
            g0s = g0_v[tt, pl.ds(0, 16)]
            g1s = g1_v[tt, pl.ds(0, 16)]

            def vec(v, carry2):
                s = v // 8
                l16 = (v % 8) * 16
                o = (g0s * r0_v[tt, s, pl.ds(l16, 16)]
                     + g1s * r1_v[tt, s, pl.ds(l16, 16)])
                o_v[tt, s, pl.ds(l16, 16)] = o
                return carry2

            jax.lax.fori_loop(0, _D // 16, vec, 0)
            return carry

        jax.lax.fori_loop(0, _CC, tok, 0)
        pltpu.sync_copy(o_v, out_hbm.at[pl.ds(base, _CC)])


def _ffn_kernel(wt_ref, we_ref, wv_ref, off_ref, cnt_ref,
                xs_ref, wg_ref, wu_ref, wd_ref, out_ref,
                x2d_ref, acc_ref):
    w = pl.program_id(0)
    f = pl.program_id(1)
    t = wt_ref[w]
    e = we_ref[w]
    first = jnp.logical_or(w == 0, t != wt_ref[jnp.maximum(w - 1, 0)])
    last = jnp.logical_and(
        f == _F - 1,
        jnp.logical_or(w == _W - 1, wt_ref[jnp.minimum(w + 1, _W - 1)] != t))

    @pl.when(jnp.logical_and(first, f == 0))
    def _():
        x2d_ref[...] = xs_ref[...].reshape(_TILE, _D)
        acc_ref[...] = jnp.zeros_like(acc_ref)

    @pl.when(wv_ref[w] == 1)
    def _():
        x = x2d_ref[...]
        a = jax.lax.dot_general(x, wg_ref[0], (((1,), (1,)), ((), ())),
                                preferred_element_type=jnp.float32)
        b = jax.lax.dot_general(x, wu_ref[0], (((1,), (1,)), ((), ())),
                                preferred_element_type=jnp.float32)
        h = (a * jax.nn.sigmoid(a)) * b
        row = t * _TILE + jax.lax.broadcasted_iota(jnp.int32, (_TILE, 1), 0)
        s = off_ref[e, 0]
        mask = jnp.logical_and(row >= s, row < s + cnt_ref[e, 0])
        hm = jnp.where(mask, h, 0.0)
        acc_ref[...] += jax.lax.dot_general(
            hm, wd_ref[0], (((1,), (1,)), ((), ())),
            preferred_element_type=jnp.float32)

    @pl.when(last)
    def _():
        out_ref[...] = acc_ref[...].reshape(_TILE, 8, 128)


def kernel(x, gate_w, gate_b, w_gate, w_up, w_down):
    xf = x.reshape(-1, x.shape[-1])
    logits, x3, g0, g1, p0, p1, off, cnt = pl.pallas_call(
        _plan_kernel,
        out_shape=(
            jax.ShapeDtypeStruct((_N, _E), jnp.float32),
            jax.ShapeDtypeStruct((_N, 8, 128), jnp.float32),
            jax.ShapeDtypeStruct((_N, 128), jnp.float32),
            jax.ShapeDtypeStruct((_N, 128), jnp.float32),
            jax.ShapeDtypeStruct((_N,), jnp.int32),
            jax.ShapeDtypeStruct((_N,), jnp.int32),
            jax.ShapeDtypeStruct((_E, 1), jnp.int32),
            jax.ShapeDtypeStruct((_E, 1), jnp.int32),
        ),
    )(xf, gate_w, gate_b.reshape(1, _E))

    wt, we, wv = pl.pallas_call(
        _worklist_kernel,
        grid_spec=pltpu.PrefetchScalarGridSpec(
            num_scalar_prefetch=2,
            grid=(1,),
            in_specs=[],
            out_specs=[
                pl.BlockSpec(memory_space=pltpu.SMEM),
                pl.BlockSpec(memory_space=pltpu.SMEM),
                pl.BlockSpec(memory_space=pltpu.SMEM),
            ],
        ),
        out_shape=(
            jax.ShapeDtypeStruct((_W,), jnp.int32),
            jax.ShapeDtypeStruct((_W,), jnp.int32),
            jax.ShapeDtypeStruct((_W,), jnp.int32),
        ),
    )(off, cnt)

    xs = _sc_dispatch(x3, p0, p1)

    outs = pl.pallas_call(
        _ffn_kernel,
        grid_spec=pltpu.PrefetchScalarGridSpec(
            num_scalar_prefetch=5,
            grid=(_W, _F),
            in_specs=[
                pl.BlockSpec((_TILE, 8, 128),
                             lambda w, f, wt, we, wv, o, c: (wt[w], 0, 0)),
                pl.BlockSpec((1, _FFT, _D),
                             lambda w, f, wt, we, wv, o, c: (we[w], f, 0)),
                pl.BlockSpec((1, _FFT, _D),
                             lambda w, f, wt, we, wv, o, c: (we[w], f, 0)),
                pl.BlockSpec((1, _D, _FFT),
                             lambda w, f, wt, we, wv, o, c: (we[w], 0, f)),
            ],
            out_specs=pl.BlockSpec(
                (_TILE, 8, 128), lambda w, f, wt, we, wv, o, c: (wt[w], 0, 0)),
            scratch_shapes=[
                pltpu.VMEM((_TILE, _D), jnp.float32),
                pltpu.VMEM((_TILE, _D), jnp.float32),
            ],
        ),
        out_shape=jax.ShapeDtypeStruct((_NK, 8, 128), jnp.float32),
    )(wt, we, wv, off, cnt, xs, w_gate, w_up, w_down)

    final = _sc_combine(outs, p0, p1, g0, g1)
    return final.reshape(x.shape), logits


# speedup vs baseline: 1.1157x; 1.1157x over previous
import functools

import jax
import jax.numpy as jnp
from jax import lax
from jax.experimental import pallas as pl
from jax.experimental.pallas import tpu as pltpu
from jax.experimental.pallas import tpu_sc as plsc

_E, _D, _FF, _K = 8, 1024, 2048, 2
_N = 2048
_NK = _N * _K
_TILE = 512
_T = _NK // _TILE
_W = _T + _E - 1
_FFT = 512
_F = _FF // _FFT

_INFO = plsc.get_sparse_core_info()
_NW = _INFO.num_cores * _INFO.num_subcores
_CHUNK = _N // _NW


def _cumsum(v, axis):
    n = v.shape[axis]
    sh = 1
    while sh < n:
        z = jnp.zeros_like(jax.lax.slice_in_dim(v, 0, sh, axis=axis))
        v = v + jnp.concatenate(
            [z, jax.lax.slice_in_dim(v, 0, n - sh, axis=axis)], axis=axis)
        sh *= 2
    return v


def _plan_kernel(x_ref, gw_ref, gb_ref, logits_ref, x3_ref,
                 g0_ref, g1_ref, p0_ref, p1_ref, off_ref, cnt_ref):
    x = x_ref[...]
    x3_ref[...] = x.reshape(_N, 8, 128)
    logits = jax.lax.dot_general(
        x, gw_ref[...], (((1,), (1,)), ((), ())),
        preferred_element_type=jnp.float32) + gb_ref[...]
    logits_ref[...] = logits
    lt = logits.T
    sub = jax.lax.broadcasted_iota(jnp.int32, lt.shape, 0)
    l0 = jnp.max(lt, axis=0, keepdims=True)
    i0 = jnp.min(jnp.where(lt == l0, sub, _E), axis=0, keepdims=True)
    masked = jnp.where(sub == i0, -jnp.inf, lt)
    l1 = jnp.max(masked, axis=0, keepdims=True)
    i1 = jnp.min(jnp.where(masked == l1, sub, _E), axis=0, keepdims=True)
    g0 = jax.nn.sigmoid(l0 - l1)
    lane = jax.lax.broadcasted_iota(jnp.int32, logits.shape, 1)
    l0c = jnp.max(logits, axis=1, keepdims=True)
    i0c = jnp.min(jnp.where(logits == l0c, lane, _E), axis=1, keepdims=True)
    mkc = jnp.where(lane == i0c, -jnp.inf, logits)
    l1c = jnp.max(mkc, axis=1, keepdims=True)
    g0c = jax.nn.sigmoid(l0c - l1c)
    g0_ref[...] = jnp.broadcast_to(g0c, (_N, 128))
    g1_ref[...] = jnp.broadcast_to(1.0 - g0c, (_N, 128))
    m0 = (sub == i0).astype(jnp.int32)
    m1 = (sub == i1).astype(jnp.int32)
    c0i = _cumsum(m0, 1)
    c1i = _cumsum(m1, 1)
    cnt0 = c0i[:, _N - 1:_N]
    cnt1 = c1i[:, _N - 1:_N]
    counts = cnt0 + cnt1
    off = _cumsum(counts, 0) - counts
    off_ref[...] = off
    cnt_ref[...] = counts
    p0 = jnp.sum(m0 * (off + c0i - m0), axis=0, keepdims=True)
    p1 = jnp.sum(m1 * (off + cnt0 + c1i - m1), axis=0, keepdims=True)
    p0_ref[...] = p0.reshape(_N)
    p1_ref[...] = p1.reshape(_N)


def _worklist_kernel(off_ref, cnt_ref, wt_ref, we_ref, wv_ref):
    def wbody(p, idx):
        t = p // _E
        e = p % _E
        s = off_ref[e, 0]
        c = cnt_ref[e, 0]
        active = jnp.logical_and(
            jnp.logical_and(s < (t + 1) * _TILE, s + c > t * _TILE), c > 0)

        @pl.when(active)
        def _():
            wt_ref[idx] = t
            we_ref[idx] = e
            wv_ref[idx] = 1

        return idx + active.astype(jnp.int32)

    nitems = jax.lax.fori_loop(0, _T * _E, wbody, 0)

    def pbody(i, carry):
        @pl.when(i >= nitems)
        def _():
            wt_ref[i] = wt_ref[nitems - 1]
            we_ref[i] = we_ref[nitems - 1]
            wv_ref[i] = 0
        return carry

    jax.lax.fori_loop(0, _W, pbody, 0)


_SC_MESH = plsc.VectorSubcoreMesh(core_axis_name="c", subcore_axis_name="s")


@functools.partial(
    pl.kernel, mesh=_SC_MESH,
    out_type=jax.ShapeDtypeStruct((_NK, 8, 128), jnp.float32),
    scratch_types=[
        pltpu.VMEM((_CHUNK,), jnp.int32),
        pltpu.VMEM((_CHUNK,), jnp.int32),
        pltpu.VMEM((_CHUNK, 8, 128), jnp.float32),
        pltpu.SemaphoreType.DMA,
    ],
)
def _sc_dispatch(x3_hbm, p0_hbm, p1_hbm, xs_hbm, i0_v, i1_v, rows_v, sem):
    wid = lax.axis_index("s") * _INFO.num_cores + lax.axis_index("c")
    base = wid * _CHUNK
    pltpu.sync_copy(p0_hbm.at[pl.ds(base, _CHUNK)], i0_v)
    pltpu.sync_copy(p1_hbm.at[pl.ds(base, _CHUNK)], i1_v)
    pltpu.async_copy(x3_hbm.at[pl.ds(base, _CHUNK)], rows_v, sem).wait()
    pltpu.sync_copy(rows_v, xs_hbm.at[i0_v])
    pltpu.sync_copy(rows_v, xs_hbm.at[i1_v])


_CC = 32


@functools.partial(
    pl.kernel, mesh=_SC_MESH,
    out_type=jax.ShapeDtypeStruct((_N, 8, 128), jnp.float32),
    scratch_types=[
        pltpu.VMEM((_CC,), jnp.int32),
        pltpu.VMEM((_CC,), jnp.int32),
        pltpu.VMEM((_CC, 128), jnp.float32),
        pltpu.VMEM((_CC, 128), jnp.float32),
        pltpu.VMEM((_CC, 8, 128), jnp.float32),
        pltpu.VMEM((_CC, 8, 128), jnp.float32),
        pltpu.VMEM((_CC, 8, 128), jnp.float32),
        pltpu.SemaphoreType.DMA,
        pltpu.SemaphoreType.DMA,
    ],
)
def _sc_combine(os_hbm, p0_hbm, p1_hbm, g0_hbm, g1_hbm, out_hbm,
                i0_v, i1_v, g0_v, g1_v, r0_v, r1_v, o_v, sem0, sem1):
    wid = lax.axis_index("s") * _INFO.num_cores + lax.axis_index("c")
    for cc in range(_CHUNK // _CC):
        base = wid * _CHUNK + cc * _CC
        pltpu.sync_copy(p0_hbm.at[pl.ds(base, _CC)], i0_v)
        pltpu.sync_copy(p1_hbm.at[pl.ds(base, _CC)], i1_v)
        pltpu.sync_copy(g0_hbm.at[pl.ds(base, _CC)], g0_v)
        pltpu.sync_copy(g1_hbm.at[pl.ds(base, _CC)], g1_v)
        c0 = pltpu.async_copy(os_hbm.at[i0_v], r0_v, sem0)
        c1 = pltpu.async_copy(os_hbm.at[i1_v], r1_v, sem1)
        c0.wait()
        c1.wait()

        def tok(tt, carry):
            g0s = g0_v[tt, pl.ds(0, 16)]
            g1s = g1_v[tt, pl.ds(0, 16)]
            for v in range(_D // 16):
                s = v // 8
                l16 = (v % 8) * 16
                o_v[tt, s, pl.ds(l16, 16)] = (
                    g0s * r0_v[tt, s, pl.ds(l16, 16)]
                    + g1s * r1_v[tt, s, pl.ds(l16, 16)])
            return carry

        jax.lax.fori_loop(0, _CC, tok, 0)
        pltpu.sync_copy(o_v, out_hbm.at[pl.ds(base, _CC)])


def _ffn_kernel(wt_ref, we_ref, wv_ref, off_ref, cnt_ref,
                xs_ref, wg_ref, wu_ref, wd_ref, out_ref,
                x2d_ref, acc_ref):
    w = pl.program_id(0)
    f = pl.program_id(1)
    t = wt_ref[w]
    e = we_ref[w]
    first = jnp.logical_or(w == 0, t != wt_ref[jnp.maximum(w - 1, 0)])
    last = jnp.logical_and(
        f == _F - 1,
        jnp.logical_or(w == _W - 1, wt_ref[jnp.minimum(w + 1, _W - 1)] != t))

    @pl.when(jnp.logical_and(first, f == 0))
    def _():
        x2d_ref[...] = xs_ref[...].reshape(_TILE, _D)
        acc_ref[...] = jnp.zeros_like(acc_ref)

    @pl.when(wv_ref[w] == 1)
    def _():
        x = x2d_ref[...]
        a = jax.lax.dot_general(x, wg_ref[0], (((1,), (1,)), ((), ())),
                                preferred_element_type=jnp.float32)
        b = jax.lax.dot_general(x, wu_ref[0], (((1,), (1,)), ((), ())),
                                preferred_element_type=jnp.float32)
        h = (a * jax.nn.sigmoid(a)) * b
        row = t * _TILE + jax.lax.broadcasted_iota(jnp.int32, (_TILE, 1), 0)
        s = off_ref[e, 0]
        mask = jnp.logical_and(row >= s, row < s + cnt_ref[e, 0])
        hm = jnp.where(mask, h, 0.0)
        acc_ref[...] += jax.lax.dot_general(
            hm, wd_ref[0], (((1,), (1,)), ((), ())),
            preferred_element_type=jnp.float32)

    @pl.when(last)
    def _():
        out_ref[...] = acc_ref[...].reshape(_TILE, 8, 128)


def kernel(x, gate_w, gate_b, w_gate, w_up, w_down):
    xf = x.reshape(-1, x.shape[-1])
    logits, x3, g0, g1, p0, p1, off, cnt = pl.pallas_call(
        _plan_kernel,
        out_shape=(
            jax.ShapeDtypeStruct((_N, _E), jnp.float32),
            jax.ShapeDtypeStruct((_N, 8, 128), jnp.float32),
            jax.ShapeDtypeStruct((_N, 128), jnp.float32),
            jax.ShapeDtypeStruct((_N, 128), jnp.float32),
            jax.ShapeDtypeStruct((_N,), jnp.int32),
            jax.ShapeDtypeStruct((_N,), jnp.int32),
            jax.ShapeDtypeStruct((_E, 1), jnp.int32),
            jax.ShapeDtypeStruct((_E, 1), jnp.int32),
        ),
    )(xf, gate_w, gate_b.reshape(1, _E))

    wt, we, wv = pl.pallas_call(
        _worklist_kernel,
        grid_spec=pltpu.PrefetchScalarGridSpec(
            num_scalar_prefetch=2,
            grid=(1,),
            in_specs=[],
            out_specs=[
                pl.BlockSpec(memory_space=pltpu.SMEM),
                pl.BlockSpec(memory_space=pltpu.SMEM),
                pl.BlockSpec(memory_space=pltpu.SMEM),
            ],
        ),
        out_shape=(
            jax.ShapeDtypeStruct((_W,), jnp.int32),
            jax.ShapeDtypeStruct((_W,), jnp.int32),
            jax.ShapeDtypeStruct((_W,), jnp.int32),
        ),
    )(off, cnt)

    xs = _sc_dispatch(x3, p0, p1)

    outs = pl.pallas_call(
        _ffn_kernel,
        grid_spec=pltpu.PrefetchScalarGridSpec(
            num_scalar_prefetch=5,
            grid=(_W, _F),
            in_specs=[
                pl.BlockSpec((_TILE, 8, 128),
                             lambda w, f, wt, we, wv, o, c: (wt[w], 0, 0)),
                pl.BlockSpec((1, _FFT, _D),
                             lambda w, f, wt, we, wv, o, c: (we[w], f, 0)),
                pl.BlockSpec((1, _FFT, _D),
                             lambda w, f, wt, we, wv, o, c: (we[w], f, 0)),
                pl.BlockSpec((1, _D, _FFT),
                             lambda w, f, wt, we, wv, o, c: (we[w], 0, f)),
            ],
            out_specs=pl.BlockSpec(
                (_TILE, 8, 128), lambda w, f, wt, we, wv, o, c: (wt[w], 0, 0)),
            scratch_shapes=[
                pltpu.VMEM((_TILE, _D), jnp.float32),
                pltpu.VMEM((_TILE, _D), jnp.float32),
            ],
        ),
        out_shape=jax.ShapeDtypeStruct((_NK, 8, 128), jnp.float32),
    )(wt, we, wv, off, cnt, xs, w_gate, w_up, w_down)

    final = _sc_combine(outs, p0, p1, g0, g1)
    return final.reshape(x.shape), logits
